# baseline (speedup 1.0000x reference)
import functools

import jax
import jax.numpy as jnp
from jax import lax
from jax.experimental import pallas as pl
from jax.experimental.pallas import tpu as pltpu
from jax.experimental.pallas import tpu_sc as plsc

BATCH = 16384
MF_DIM = 16
MLP_DIM = 8

_NC = 2
_NS = 16
_NW = _NC * _NS
_EPW = BATCH // _NW
_GRP = 8
_NGRP = _EPW // _GRP
_RW = _GRP * 128


def _sc_gather_body(uid_hbm, mid_hbm, mfu_t, mfi_t, mlu_t, mli_t,
                    out_mfu, out_mfi, out_mlu, out_mli,
                    vidx_u, vidx_m,
                    ra_mfu, ra_mfi, ra_mlu, ra_mli,
                    rb_mfu, rb_mfi, rb_mlu, rb_mli,
                    bmfu, bmfi, bmlu, bmli, sem_a, sem_b):
    wid = lax.axis_index("s") * _NC + lax.axis_index("c")
    base = pl.multiple_of(wid * _EPW, 128)
    pltpu.sync_copy(uid_hbm.at[pl.ds(base, _EPW)], vidx_u)
    pltpu.sync_copy(mid_hbm.at[pl.ds(base, _EPW)], vidx_m)

    rows = lax.iota(jnp.int32, 16)
    rows8 = jnp.bitwise_and(rows, 7)
    mask8 = rows < 8

    def fire(uvec, mvec, half, ring, sem):
        r_mfu, r_mfi, r_mlu, r_mli = ring
        for j in range(_GRP):
            u = uvec[half * _GRP + j]
            m = mvec[half * _GRP + j]
            cu = pl.multiple_of((u >> 7) * 128, 128)
            cm = pl.multiple_of((m >> 7) * 128, 128)
            dst = pl.ds(j * 128, 128)
            pltpu.async_copy(mfu_t.at[:, pl.ds(cu, 128)], r_mfu.at[:, dst], sem)
            pltpu.async_copy(mfi_t.at[:, pl.ds(cm, 128)], r_mfi.at[:, dst], sem)
            pltpu.async_copy(mlu_t.at[:, pl.ds(cu, 128)], r_mlu.at[:, dst], sem)
            pltpu.async_copy(mli_t.at[:, pl.ds(cm, 128)], r_mli.at[:, dst], sem)

    def wait(ring, sem):
        r_mfu, r_mfi, r_mlu, r_mli = ring
        pltpu.make_async_copy(out_mfu.at[:, pl.ds(0, _RW)], r_mfu, sem).wait()
        pltpu.make_async_copy(out_mfi.at[:, pl.ds(0, _RW)], r_mfi, sem).wait()
        pltpu.make_async_copy(out_mlu.at[:, pl.ds(0, _RW)], r_mlu, sem).wait()
        pltpu.make_async_copy(out_mli.at[:, pl.ds(0, _RW)], r_mli, sem).wait()

    def extract(uvec, mvec, g, half, ring):
        r_mfu, r_mfi, r_mlu, r_mli = ring
        for j in range(_GRP):
            i = g * _GRP + j
            u = uvec[half * _GRP + j]
            m = mvec[half * _GRP + j]
            cols_u = jnp.full((16,), j * 128 + jnp.bitwise_and(u, 127), jnp.int32)
            cols_m = jnp.full((16,), j * 128 + jnp.bitwise_and(m, 127), jnp.int32)
            outc = jnp.full((16,), i, jnp.int32)
            v = plsc.load_gather(r_mfu, [rows, cols_u])
            plsc.store_scatter(bmfu, [rows, outc], v)
            v = plsc.load_gather(r_mfi, [rows, cols_m])
            plsc.store_scatter(bmfi, [rows, outc], v)
            v = plsc.load_gather(r_mlu, [rows8, cols_u], mask=mask8)
            plsc.store_scatter(bmlu, [rows8, outc], v, mask=mask8)
            v = plsc.load_gather(r_mli, [rows8, cols_m], mask=mask8)
            plsc.store_scatter(bmli, [rows8, outc], v, mask=mask8)

    ring_a = (ra_mfu, ra_mfi, ra_mlu, ra_mli)
    ring_b = (rb_mfu, rb_mfi, rb_mlu, rb_mli)

    uvec0 = vidx_u[pl.ds(0, 16)]
    mvec0 = vidx_m[pl.ds(0, 16)]
    fire(uvec0, mvec0, 0, ring_a, sem_a)

    def body(t, carry):
        g = 2 * t
        uvec = vidx_u[pl.ds(t * 16, 16)]
        mvec = vidx_m[pl.ds(t * 16, 16)]
        fire(uvec, mvec, 1, ring_b, sem_b)
        wait(ring_a, sem_a)
        extract(uvec, mvec, g, 0, ring_a)

        @pl.when(t < _NGRP // 2 - 1)
        def _():
            uvec2 = vidx_u[pl.ds(t * 16 + 16, 16)]
            mvec2 = vidx_m[pl.ds(t * 16 + 16, 16)]
            fire(uvec2, mvec2, 0, ring_a, sem_a)

        wait(ring_b, sem_b)
        extract(uvec, mvec, g + 1, 1, ring_b)
        return carry

    lax.fori_loop(0, _NGRP // 2, body, 0)

    out_slc = pl.ds(base, _EPW)
    pltpu.sync_copy(bmfu, out_mfu.at[:, out_slc])
    pltpu.sync_copy(bmfi, out_mfi.at[:, out_slc])
    pltpu.sync_copy(bmlu, out_mlu.at[:, out_slc])
    pltpu.sync_copy(bmli, out_mli.at[:, out_slc])


_sc_gather = functools.partial(
    pl.kernel,
    mesh=plsc.VectorSubcoreMesh(core_axis_name="c", subcore_axis_name="s"),
    compiler_params=pltpu.CompilerParams(use_tc_tiling_on_sc=True,
                                         needs_layout_passes=False),
    out_type=[
        jax.ShapeDtypeStruct((MF_DIM, BATCH), jnp.float32),
        jax.ShapeDtypeStruct((MF_DIM, BATCH), jnp.float32),
        jax.ShapeDtypeStruct((MLP_DIM, BATCH), jnp.float32),
        jax.ShapeDtypeStruct((MLP_DIM, BATCH), jnp.float32),
    ],
    scratch_types=[
        pltpu.VMEM((_EPW,), jnp.int32),
        pltpu.VMEM((_EPW,), jnp.int32),
        pltpu.VMEM((MF_DIM, _RW), jnp.float32),
        pltpu.VMEM((MF_DIM, _RW), jnp.float32),
        pltpu.VMEM((MLP_DIM, _RW), jnp.float32),
        pltpu.VMEM((MLP_DIM, _RW), jnp.float32),
        pltpu.VMEM((MF_DIM, _RW), jnp.float32),
        pltpu.VMEM((MF_DIM, _RW), jnp.float32),
        pltpu.VMEM((MLP_DIM, _RW), jnp.float32),
        pltpu.VMEM((MLP_DIM, _RW), jnp.float32),
        pltpu.VMEM((MF_DIM, _EPW), jnp.float32),
        pltpu.VMEM((MF_DIM, _EPW), jnp.float32),
        pltpu.VMEM((MLP_DIM, _EPW), jnp.float32),
        pltpu.VMEM((MLP_DIM, _EPW), jnp.float32),
        pltpu.SemaphoreType.DMA,
        pltpu.SemaphoreType.DMA,
    ],
)(_sc_gather_body)


def _tc_mlp_body(mfu, mfi, mlu, mli, w1u, w1i, b1, w2, b2, wl, bl,
                 w2l_mf, w2l_mlp, b2l, out):
    h1 = jnp.maximum(
        jnp.dot(w1u[...], mlu[...], preferred_element_type=jnp.float32)
        + jnp.dot(w1i[...], mli[...], preferred_element_type=jnp.float32)
        + b1[...], 0.0)
    h2 = jnp.maximum(
        jnp.dot(w2[...], h1, preferred_element_type=jnp.float32) + b2[...], 0.0)
    mlp_vec = jnp.dot(wl[...], h2, preferred_element_type=jnp.float32) + bl[...]
    mf_vec = mfu[...] * mfi[...]
    z = (jnp.dot(w2l_mf[...], mf_vec, preferred_element_type=jnp.float32)
         + jnp.dot(w2l_mlp[...], mlp_vec, preferred_element_type=jnp.float32)
         + b2l[...])
    out[...] = jax.nn.sigmoid(z)


_TC_BLOCK = 16384


def _tc_mlp(mfu, mfi, mlu, mli, w1u, w1i, b1, w2, b2, wl, bl,
            w2l_mf, w2l_mlp, b2l):
    nblk = BATCH // _TC_BLOCK
    data_spec = lambda d: pl.BlockSpec((d, _TC_BLOCK), lambda i: (0, i))
    full_spec = lambda a: pl.BlockSpec(a.shape, lambda i: (0,) * a.ndim)
    return pl.pallas_call(
        _tc_mlp_body,
        grid=(nblk,),
        in_specs=[
            data_spec(MF_DIM), data_spec(MF_DIM),
            data_spec(MLP_DIM), data_spec(MLP_DIM),
            full_spec(w1u), full_spec(w1i), full_spec(b1),
            full_spec(w2), full_spec(b2), full_spec(wl), full_spec(bl),
            full_spec(w2l_mf), full_spec(w2l_mlp), full_spec(b2l),
        ],
        out_specs=pl.BlockSpec((1, _TC_BLOCK), lambda i: (0, i)),
        out_shape=jax.ShapeDtypeStruct((1, BATCH), jnp.float32),
    )(mfu, mfi, mlu, mli, w1u, w1i, b1, w2, b2, wl, bl,
      w2l_mf, w2l_mlp, b2l)


def kernel(uid, mid, mf_user, mf_item, mlp_user, mlp_item,
           W1, b1, W2, b2, Wl, bl, W2l, b2l):
    mfu_t, mfi_t, mlu_t, mli_t = _sc_gather(
        uid, mid, mf_user.T, mf_item.T, mlp_user.T, mlp_item.T)
    out = _tc_mlp(
        mfu_t, mfi_t, mlu_t, mli_t,
        W1[:, :MLP_DIM], W1[:, MLP_DIM:], b1.reshape(-1, 1),
        W2, b2.reshape(-1, 1), Wl, bl.reshape(-1, 1),
        W2l[:, :MF_DIM], W2l[:, MF_DIM:], b2l.reshape(1, 1),
    )
    return out.reshape(BATCH)


# speedup vs baseline: 1.0076x; 1.0076x over previous
import functools

import jax
import jax.numpy as jnp
from jax import lax
from jax.experimental import pallas as pl
from jax.experimental.pallas import tpu as pltpu
from jax.experimental.pallas import tpu_sc as plsc

BATCH = 16384
MF_DIM = 16
MLP_DIM = 8

_NC = 2
_NS = 16
_NW = _NC * _NS
_EPW = BATCH // _NW
_GRP = 8
_NGRP = _EPW // _GRP
_RW = _GRP * 128


def _sc_gather_body(uid_hbm, mid_hbm, mfu_t, mfi_t, mlu_t, mli_t,
                    out_mfu, out_mfi, out_mlu, out_mli,
                    vidx_u, vidx_m,
                    ra_mfu, ra_mfi, ra_mlu, ra_mli,
                    rb_mfu, rb_mfi, rb_mlu, rb_mli,
                    bmfu, bmfi, bmlu, bmli, sem_a, sem_b):
    wid = lax.axis_index("s") * _NC + lax.axis_index("c")
    base = pl.multiple_of(wid * _EPW, 128)
    pltpu.sync_copy(uid_hbm.at[pl.ds(base, _EPW)], vidx_u)
    pltpu.sync_copy(mid_hbm.at[pl.ds(base, _EPW)], vidx_m)

    rows = lax.iota(jnp.int32, 16)
    rows8 = jnp.bitwise_and(rows, 7)
    mask8 = rows < 8

    def fire(uvec, mvec, half, ring, sem):
        r_mfu, r_mfi, r_mlu, r_mli = ring
        for j in range(_GRP):
            u = uvec[half * _GRP + j]
            m = mvec[half * _GRP + j]
            cu = pl.multiple_of((u >> 7) * 128, 128)
            cm = pl.multiple_of((m >> 7) * 128, 128)
            dst = pl.ds(j * 128, 128)
            pltpu.async_copy(mfu_t.at[:, pl.ds(cu, 128)], r_mfu.at[:, dst], sem)
            pltpu.async_copy(mfi_t.at[:, pl.ds(cm, 128)], r_mfi.at[:, dst], sem)
            pltpu.async_copy(mlu_t.at[:, pl.ds(cu, 128)], r_mlu.at[:, dst], sem)
            pltpu.async_copy(mli_t.at[:, pl.ds(cm, 128)], r_mli.at[:, dst], sem)

    def wait(ring, sem):
        r_mfu, r_mfi, r_mlu, r_mli = ring
        pltpu.make_async_copy(out_mfu.at[:, pl.ds(0, _RW)], r_mfu, sem).wait()
        pltpu.make_async_copy(out_mfi.at[:, pl.ds(0, _RW)], r_mfi, sem).wait()
        pltpu.make_async_copy(out_mlu.at[:, pl.ds(0, _RW)], r_mlu, sem).wait()
        pltpu.make_async_copy(out_mli.at[:, pl.ds(0, _RW)], r_mli, sem).wait()

    def extract(uvec, mvec, g, half, ring):
        r_mfu, r_mfi, r_mlu, r_mli = ring
        for j in range(_GRP):
            i = g * _GRP + j
            u = uvec[half * _GRP + j]
            m = mvec[half * _GRP + j]
            cols_u = jnp.full((16,), j * 128 + jnp.bitwise_and(u, 127), jnp.int32)
            cols_m = jnp.full((16,), j * 128 + jnp.bitwise_and(m, 127), jnp.int32)
            outc = jnp.full((16,), i, jnp.int32)
            v = plsc.load_gather(r_mfu, [rows, cols_u])
            plsc.store_scatter(bmfu, [rows, outc], v)
            v = plsc.load_gather(r_mfi, [rows, cols_m])
            plsc.store_scatter(bmfi, [rows, outc], v)
            v = plsc.load_gather(r_mlu, [rows8, cols_u], mask=mask8)
            plsc.store_scatter(bmlu, [rows8, outc], v, mask=mask8)
            v = plsc.load_gather(r_mli, [rows8, cols_m], mask=mask8)
            plsc.store_scatter(bmli, [rows8, outc], v, mask=mask8)

    ring_a = (ra_mfu, ra_mfi, ra_mlu, ra_mli)
    ring_b = (rb_mfu, rb_mfi, rb_mlu, rb_mli)

    uvec0 = vidx_u[pl.ds(0, 16)]
    mvec0 = vidx_m[pl.ds(0, 16)]
    fire(uvec0, mvec0, 0, ring_a, sem_a)

    def body(t, carry):
        g = 2 * t
        uvec = vidx_u[pl.ds(t * 16, 16)]
        mvec = vidx_m[pl.ds(t * 16, 16)]
        fire(uvec, mvec, 1, ring_b, sem_b)
        wait(ring_a, sem_a)
        extract(uvec, mvec, g, 0, ring_a)

        @pl.when(t < _NGRP // 2 - 1)
        def _():
            uvec2 = vidx_u[pl.ds(t * 16 + 16, 16)]
            mvec2 = vidx_m[pl.ds(t * 16 + 16, 16)]
            fire(uvec2, mvec2, 0, ring_a, sem_a)

        wait(ring_b, sem_b)
        extract(uvec, mvec, g + 1, 1, ring_b)
        return carry

    lax.fori_loop(0, _NGRP // 2, body, 0)

    out_slc = pl.ds(base, _EPW)
    pltpu.sync_copy(bmfu, out_mfu.at[:, out_slc])
    pltpu.sync_copy(bmfi, out_mfi.at[:, out_slc])
    pltpu.sync_copy(bmlu, out_mlu.at[:, out_slc])
    pltpu.sync_copy(bmli, out_mli.at[:, out_slc])


_sc_gather = functools.partial(
    pl.kernel,
    mesh=plsc.VectorSubcoreMesh(core_axis_name="c", subcore_axis_name="s"),
    compiler_params=pltpu.CompilerParams(use_tc_tiling_on_sc=True,
                                         needs_layout_passes=False),
    out_type=[
        jax.ShapeDtypeStruct((MF_DIM, BATCH), jnp.float32),
        jax.ShapeDtypeStruct((MF_DIM, BATCH), jnp.float32),
        jax.ShapeDtypeStruct((MLP_DIM, BATCH), jnp.float32),
        jax.ShapeDtypeStruct((MLP_DIM, BATCH), jnp.float32),
    ],
    scratch_types=[
        pltpu.VMEM((_EPW,), jnp.int32),
        pltpu.VMEM((_EPW,), jnp.int32),
        pltpu.VMEM((MF_DIM, _RW), jnp.float32),
        pltpu.VMEM((MF_DIM, _RW), jnp.float32),
        pltpu.VMEM((MLP_DIM, _RW), jnp.float32),
        pltpu.VMEM((MLP_DIM, _RW), jnp.float32),
        pltpu.VMEM((MF_DIM, _RW), jnp.float32),
        pltpu.VMEM((MF_DIM, _RW), jnp.float32),
        pltpu.VMEM((MLP_DIM, _RW), jnp.float32),
        pltpu.VMEM((MLP_DIM, _RW), jnp.float32),
        pltpu.VMEM((MF_DIM, _EPW), jnp.float32),
        pltpu.VMEM((MF_DIM, _EPW), jnp.float32),
        pltpu.VMEM((MLP_DIM, _EPW), jnp.float32),
        pltpu.VMEM((MLP_DIM, _EPW), jnp.float32),
        pltpu.SemaphoreType.DMA,
        pltpu.SemaphoreType.DMA,
    ],
)(_sc_gather_body)


def _tc_mlp_body(mfu, mfi, mlu, mli, w1u, w1i, b1, w2, b2, wl, bl,
                 w2l_mf, w2l_mlp, b2l, out):
    h1 = jnp.maximum(
        jnp.dot(w1u[...], mlu[...], preferred_element_type=jnp.float32)
        + jnp.dot(w1i[...], mli[...], preferred_element_type=jnp.float32)
        + b1[...], 0.0)
    h2 = jnp.maximum(
        jnp.dot(w2[...], h1, preferred_element_type=jnp.float32) + b2[...], 0.0)
    mlp_vec = jnp.dot(wl[...], h2, preferred_element_type=jnp.float32) + bl[...]
    mf_vec = mfu[...] * mfi[...]
    z = (jnp.dot(w2l_mf[...], mf_vec, preferred_element_type=jnp.float32)
         + jnp.dot(w2l_mlp[...], mlp_vec, preferred_element_type=jnp.float32)
         + b2l[...])
    out[...] = jax.nn.sigmoid(z)


_TC_BLOCK = 8192


def _tc_mlp(mfu, mfi, mlu, mli, w1u, w1i, b1, w2, b2, wl, bl,
            w2l_mf, w2l_mlp, b2l):
    nblk = BATCH // _TC_BLOCK
    data_spec = lambda d: pl.BlockSpec((d, _TC_BLOCK), lambda i: (0, i))
    full_spec = lambda a: pl.BlockSpec(a.shape, lambda i: (0,) * a.ndim)
    return pl.pallas_call(
        _tc_mlp_body,
        grid=(nblk,),
        in_specs=[
            data_spec(MF_DIM), data_spec(MF_DIM),
            data_spec(MLP_DIM), data_spec(MLP_DIM),
            full_spec(w1u), full_spec(w1i), full_spec(b1),
            full_spec(w2), full_spec(b2), full_spec(wl), full_spec(bl),
            full_spec(w2l_mf), full_spec(w2l_mlp), full_spec(b2l),
        ],
        out_specs=pl.BlockSpec((1, _TC_BLOCK), lambda i: (0, i)),
        out_shape=jax.ShapeDtypeStruct((1, BATCH), jnp.float32),
    )(mfu, mfi, mlu, mli, w1u, w1i, b1, w2, b2, wl, bl,
      w2l_mf, w2l_mlp, b2l)


def kernel(uid, mid, mf_user, mf_item, mlp_user, mlp_item,
           W1, b1, W2, b2, Wl, bl, W2l, b2l):
    mfu_t, mfi_t, mlu_t, mli_t = _sc_gather(
        uid, mid, mf_user.T, mf_item.T, mlp_user.T, mlp_item.T)
    out = _tc_mlp(
        mfu_t, mfi_t, mlu_t, mli_t,
        W1[:, :MLP_DIM], W1[:, MLP_DIM:], b1.reshape(-1, 1),
        W2, b2.reshape(-1, 1), Wl, bl.reshape(-1, 1),
        W2l[:, :MF_DIM], W2l[:, MF_DIM:], b2l.reshape(1, 1),
    )
    return out.reshape(BATCH)
